# baseline (speedup 1.0000x reference)
import functools

import jax
import jax.numpy as jnp
from jax import lax
from jax.experimental import pallas as pl
from jax.experimental.pallas import tpu as pltpu
from jax.experimental.pallas import tpu_sc as plsc

CHUNK = 1600
BAND = CHUNK // 4
SUBS = [(0, 128), (128, 128), (256, 128), (384, 16)]
BLK = 16


def _make_gather(batch: int, hist: int, n_dim: int):
    info = plsc.get_sparse_core_info()
    nc, ns = info.num_cores, info.num_subcores
    nw = nc * ns
    total = batch * hist
    per_w = total // nw
    n_chunks = per_w // CHUNK
    n_pairs = n_chunks // 2
    pack = 128 // n_dim
    out_rows = total // pack
    b_per_pair = 2 * CHUNK // hist

    mesh = plsc.VectorSubcoreMesh(core_axis_name="c", subcore_axis_name="s")

    @functools.partial(
        pl.kernel,
        mesh=mesh,
        out_type=jax.ShapeDtypeStruct((out_rows, 128), jnp.float32),
        scratch_types=[
            pltpu.VMEM((b_per_pair, hist), jnp.int32),
            pltpu.VMEM((2 * CHUNK,), jnp.int32),
            pltpu.VMEM((pack, BAND, n_dim), jnp.float32),
            pltpu.VMEM((pack, BAND, n_dim), jnp.float32),
            pltpu.SemaphoreType.DMA,
            pltpu.SemaphoreType.DMA,
            pltpu.SemaphoreType.DMA,
            pltpu.SemaphoreType.DMA,
        ],
        compiler_params=pltpu.CompilerParams(use_tc_tiling_on_sc=False,
                                             needs_layout_passes=False),
    )
    def gather_kernel(idx_hbm, table_hbm, out_hbm, idx_raw, idx_band,
                      rows0, rows1, semg0, semg1, semw0, semw1):
        wid = lax.axis_index("s") * nc + lax.axis_index("c")
        w_b0 = wid * (per_w // hist)
        w_r0 = wid * (per_w // pack)

        def load_and_band_idx(pair):
            pltpu.sync_copy(
                idx_hbm.at[pl.ds(w_b0 + pair * b_per_pair, b_per_pair)],
                idx_raw)

            iota4 = 4 * lax.iota(jnp.int32, 16)
            for j in range(2 * CHUNK // 16):
                base = j * 16
                c, dloc = divmod(base, CHUNK)
                q, k0 = divmod(dloc, BAND)
                p0 = 4 * k0 + q
                g0, h0 = divmod(p0, hist)
                h_raw = h0 + iota4
                wrap = jnp.where(h_raw >= hist, 1, 0)
                g_vec = (c * (CHUNK // hist) + g0) + wrap
                h_vec = h_raw - hist * wrap
                vals = plsc.load_gather(idx_raw, [g_vec, h_vec])
                idx_band[pl.ds(base, 16)] = vals

        def fire_gathers(rows_v, sem, base):
            copies = []
            for q in range(pack):
                for off, n in SUBS:
                    copies.append(pltpu.async_copy(
                        table_hbm.at[
                            idx_band.at[pl.ds(base + q * BAND + off, n)]],
                        rows_v.at[q, pl.ds(off, n)], sem))
            return copies

        def fire_writes(rows_v, sem, chunk):
            r0 = w_r0 + chunk * BAND
            for q in range(pack):
                pltpu.async_copy(
                    rows_v.at[q],
                    out_hbm.at[pl.ds(r0, BAND), pl.ds(q * n_dim, n_dim)], sem)

        def wait_writes(rows_v, sem):
            for q in range(pack):
                pltpu.make_async_copy(
                    rows_v.at[q],
                    out_hbm.at[pl.ds(0, BAND), pl.ds(q * n_dim, n_dim)],
                    sem).wait()

        load_and_band_idx(0)
        g0 = fire_gathers(rows0, semg0, 0)
        g1 = fire_gathers(rows1, semg1, CHUNK)
        for c in g0:
            c.wait()
        fire_writes(rows0, semw0, 0)
        for c in g1:
            c.wait()
        fire_writes(rows1, semw1, 1)

        def pair_body(p, carry):
            wait_writes(rows0, semw0)
            wait_writes(rows1, semw1)
            load_and_band_idx(p)
            g0 = fire_gathers(rows0, semg0, 0)
            g1 = fire_gathers(rows1, semg1, CHUNK)
            for c in g0:
                c.wait()
            fire_writes(rows0, semw0, 2 * p)
            for c in g1:
                c.wait()
            fire_writes(rows1, semw1, 2 * p + 1)
            return carry

        lax.fori_loop(1, n_pairs, pair_body, 0)
        wait_writes(rows0, semw0)
        wait_writes(rows1, semw1)

    return gather_kernel


def _repack_body(in_ref, out_ref):
    x = in_ref[...]
    n_dim = out_ref.shape[-1]
    parts = [x[:, i * n_dim:(i + 1) * n_dim] for i in range(128 // n_dim)]
    y = jnp.stack(parts, axis=1)
    out_ref[...] = y.reshape(out_ref.shape)


def _make_repack(batch: int, hist: int, n_dim: int):
    rows_per_blk = BLK * hist * n_dim // 128
    return pl.pallas_call(
        _repack_body,
        grid=(batch // BLK,),
        in_specs=[pl.BlockSpec((rows_per_blk, 128), lambda i: (i, 0))],
        out_specs=pl.BlockSpec((BLK, hist, n_dim), lambda i: (i, 0, 0)),
        out_shape=jax.ShapeDtypeStruct((batch, hist, n_dim), jnp.float32),
    )


def kernel(node_ids, emb_table):
    b, h = node_ids.shape
    n_nodes, n_dim = emb_table.shape
    out2d = _make_gather(b, h, n_dim)(node_ids.astype(jnp.int32), emb_table)
    return _make_repack(b, h, n_dim)(out2d)


# speedup vs baseline: 1.8497x; 1.8497x over previous
import functools

import jax
import jax.numpy as jnp
from jax import lax
from jax.experimental import pallas as pl
from jax.experimental.pallas import tpu as pltpu
from jax.experimental.pallas import tpu_sc as plsc

G = 8
S0 = 128
S1 = 72


def _make_gather(batch: int, hist: int, n_dim: int):
    info = plsc.get_sparse_core_info()
    nc, ns = info.num_cores, info.num_subcores
    nw = nc * ns
    b_per_w = batch // nw
    n_chunks = b_per_w // G
    n_pairs = n_chunks // 2

    mesh = plsc.VectorSubcoreMesh(core_axis_name="c", subcore_axis_name="s")

    @functools.partial(
        pl.kernel,
        mesh=mesh,
        out_type=jax.ShapeDtypeStruct((batch, hist, n_dim), jnp.float32),
        scratch_types=[
            pltpu.VMEM((2 * G, hist), jnp.int32),
            pltpu.VMEM((G, hist, n_dim), jnp.float32),
            pltpu.VMEM((G, hist, n_dim), jnp.float32),
            pltpu.SemaphoreType.DMA,
            pltpu.SemaphoreType.DMA,
            pltpu.SemaphoreType.DMA,
            pltpu.SemaphoreType.DMA,
        ],
        compiler_params=pltpu.CompilerParams(use_tc_tiling_on_sc=False),
    )
    def gather_kernel(idx_hbm, table_hbm, out_hbm, idx_v, rows0, rows1,
                      semg0, semg1, semw0, semw1):
        wid = lax.axis_index("s") * nc + lax.axis_index("c")
        w_b0 = wid * b_per_w

        def load_idx(pair):
            pltpu.sync_copy(idx_hbm.at[pl.ds(w_b0 + pair * 2 * G, 2 * G)],
                            idx_v)

        def fire_gathers(rows_v, sem, g0):
            copies = []
            for g in range(G):
                copies.append(pltpu.async_copy(
                    table_hbm.at[idx_v.at[g0 + g, pl.ds(0, S0)]],
                    rows_v.at[g, pl.ds(0, S0)], sem))
                copies.append(pltpu.async_copy(
                    table_hbm.at[idx_v.at[g0 + g, pl.ds(S0, S1)]],
                    rows_v.at[g, pl.ds(S0, S1)], sem))
            return copies

        def fire_write(rows_v, sem, chunk):
            return pltpu.async_copy(
                rows_v, out_hbm.at[pl.ds(w_b0 + chunk * G, G)], sem)

        def wait_write(rows_v, sem):
            pltpu.make_async_copy(
                rows_v, out_hbm.at[pl.ds(0, G)], sem).wait()

        load_idx(0)
        g0 = fire_gathers(rows0, semg0, 0)
        g1 = fire_gathers(rows1, semg1, G)
        for c in g0:
            c.wait()
        fire_write(rows0, semw0, 0)
        for c in g1:
            c.wait()
        fire_write(rows1, semw1, 1)

        def pair_body(p, carry):
            wait_write(rows0, semw0)
            load_idx(p)
            g0 = fire_gathers(rows0, semg0, 0)
            wait_write(rows1, semw1)
            g1 = fire_gathers(rows1, semg1, G)
            for c in g0:
                c.wait()
            fire_write(rows0, semw0, 2 * p)
            for c in g1:
                c.wait()
            fire_write(rows1, semw1, 2 * p + 1)
            return carry

        lax.fori_loop(1, n_pairs, pair_body, 0)
        wait_write(rows0, semw0)
        wait_write(rows1, semw1)

    return gather_kernel


def kernel(node_ids, emb_table):
    b, h = node_ids.shape
    n_nodes, n_dim = emb_table.shape
    return _make_gather(b, h, n_dim)(node_ids.astype(jnp.int32), emb_table)


# speedup vs baseline: 1.8509x; 1.0007x over previous
import functools

import jax
import jax.numpy as jnp
from jax import lax
from jax.experimental import pallas as pl
from jax.experimental.pallas import tpu as pltpu
from jax.experimental.pallas import tpu_sc as plsc

G = 8
S0 = 128
S1 = 72


def _make_gather(batch: int, hist: int, n_dim: int):
    info = plsc.get_sparse_core_info()
    nc, ns = info.num_cores, info.num_subcores
    nw = nc * ns
    b_per_w = batch // nw
    n_chunks = b_per_w // G
    n_pairs = n_chunks // 2

    mesh = plsc.VectorSubcoreMesh(core_axis_name="c", subcore_axis_name="s")

    @functools.partial(
        pl.kernel,
        mesh=mesh,
        out_type=jax.ShapeDtypeStruct((batch, hist, n_dim), jnp.float32),
        scratch_types=[
            pltpu.VMEM((2 * G, hist), jnp.int32),
            pltpu.VMEM((G, hist, n_dim), jnp.float32),
            pltpu.VMEM((G, hist, n_dim), jnp.float32),
            pltpu.SemaphoreType.DMA,
            pltpu.SemaphoreType.DMA,
            pltpu.SemaphoreType.DMA,
            pltpu.SemaphoreType.DMA,
        ],
        compiler_params=pltpu.CompilerParams(use_tc_tiling_on_sc=False,
                                             needs_layout_passes=False),
    )
    def gather_kernel(idx_hbm, table_hbm, out_hbm, idx_v, rows0, rows1,
                      semg0, semg1, semw0, semw1):
        wid = lax.axis_index("s") * nc + lax.axis_index("c")
        w_b0 = wid * b_per_w

        def load_idx(pair):
            pltpu.sync_copy(idx_hbm.at[pl.ds(w_b0 + pair * 2 * G, 2 * G)],
                            idx_v)

        def fire_gathers(rows_v, sem, g0):
            copies = []
            for g in range(G):
                copies.append(pltpu.async_copy(
                    table_hbm.at[idx_v.at[g0 + g, pl.ds(0, S0)]],
                    rows_v.at[g, pl.ds(0, S0)], sem))
                copies.append(pltpu.async_copy(
                    table_hbm.at[idx_v.at[g0 + g, pl.ds(S0, S1)]],
                    rows_v.at[g, pl.ds(S0, S1)], sem))
            return copies

        def fire_write(rows_v, sem, chunk):
            return pltpu.async_copy(
                rows_v, out_hbm.at[pl.ds(w_b0 + chunk * G, G)], sem)

        def wait_write(rows_v, sem):
            pltpu.make_async_copy(
                rows_v, out_hbm.at[pl.ds(0, G)], sem).wait()

        load_idx(0)
        g0 = fire_gathers(rows0, semg0, 0)
        g1 = fire_gathers(rows1, semg1, G)
        for c in g0:
            c.wait()
        fire_write(rows0, semw0, 0)
        for c in g1:
            c.wait()
        fire_write(rows1, semw1, 1)

        def pair_body(p, carry):
            wait_write(rows0, semw0)
            load_idx(p)
            g0 = fire_gathers(rows0, semg0, 0)
            wait_write(rows1, semw1)
            g1 = fire_gathers(rows1, semg1, G)
            for c in g0:
                c.wait()
            fire_write(rows0, semw0, 2 * p)
            for c in g1:
                c.wait()
            fire_write(rows1, semw1, 2 * p + 1)
            return carry

        lax.fori_loop(1, n_pairs, pair_body, 0)
        wait_write(rows0, semw0)
        wait_write(rows1, semw1)

    return gather_kernel


def kernel(node_ids, emb_table):
    b, h = node_ids.shape
    n_nodes, n_dim = emb_table.shape
    return _make_gather(b, h, n_dim)(node_ids.astype(jnp.int32), emb_table)
